# baseline (speedup 1.0000x reference)
import functools

import jax
import jax.numpy as jnp
from jax.experimental import pallas as pl

_ROWS = 16384
_HID = 2048
_EXPERTS = 64
_BR = 512


def _router_kernel(x_ref, w_ref, val_ref, idx_ref):
    lg = jax.lax.dot_general(
        w_ref[...], x_ref[...], (((1,), (1,)), ((), ())),
        preferred_element_type=jnp.float32,
    )
    iota = jax.lax.broadcasted_iota(jnp.int32, lg.shape, 0)
    m1 = jnp.max(lg, axis=0, keepdims=True)
    i1 = jnp.min(jnp.where(lg == m1, iota, _EXPERTS), axis=0, keepdims=True)
    masked = jnp.where(iota == i1, -jnp.inf, lg)
    m2 = jnp.max(masked, axis=0, keepdims=True)
    i2 = jnp.min(jnp.where(masked == m2, iota, _EXPERTS), axis=0, keepdims=True)
    e2 = jnp.exp(m2 - m1)
    inv = 1.0 / (1.0 + e2)
    val_ref[...] = jnp.concatenate([inv, e2 * inv], axis=0).T
    idx_ref[...] = jnp.concatenate([i1, i2], axis=0).T


@jax.jit
def kernel(hidden_states, weight):
    grid = (_ROWS // _BR,)
    vals, idx = pl.pallas_call(
        _router_kernel,
        grid=grid,
        in_specs=[
            pl.BlockSpec((_BR, _HID), lambda i: (i, 0)),
            pl.BlockSpec((_EXPERTS, _HID), lambda i: (0, 0)),
        ],
        out_specs=[
            pl.BlockSpec((_BR, 2), lambda i: (i, 0)),
            pl.BlockSpec((_BR, 2), lambda i: (i, 0)),
        ],
        out_shape=[
            jax.ShapeDtypeStruct((_ROWS, 2), jnp.float32),
            jax.ShapeDtypeStruct((_ROWS, 2), jnp.int32),
        ],
    )(hidden_states, weight)
    return (vals, idx)


# speedup vs baseline: 1.5802x; 1.5802x over previous
import functools

import jax
import jax.numpy as jnp
from jax.experimental import pallas as pl

_ROWS = 16384
_HID = 2048
_EXPERTS = 64
_BR = 2048


def _router_kernel(x_ref, w_ref, val_ref, idx_ref):
    lg = jax.lax.dot_general(
        w_ref[...], x_ref[...], (((1,), (1,)), ((), ())),
        preferred_element_type=jnp.float32,
    )
    iota = jax.lax.broadcasted_iota(jnp.int32, lg.shape, 0)
    m1 = jnp.max(lg, axis=0, keepdims=True)
    i1 = jnp.min(jnp.where(lg == m1, iota, _EXPERTS), axis=0, keepdims=True)
    masked = jnp.where(iota == i1, -jnp.inf, lg)
    m2 = jnp.max(masked, axis=0, keepdims=True)
    i2 = jnp.min(jnp.where(masked == m2, iota, _EXPERTS), axis=0, keepdims=True)
    e2 = jnp.exp(m2 - m1)
    inv = 1.0 / (1.0 + e2)
    val_ref[...] = jnp.concatenate([inv, e2 * inv], axis=0)
    idx_ref[...] = jnp.concatenate([i1, i2], axis=0)


@jax.jit
def kernel(hidden_states, weight):
    grid = (_ROWS // _BR,)
    vals, idx = pl.pallas_call(
        _router_kernel,
        grid=grid,
        in_specs=[
            pl.BlockSpec((_BR, _HID), lambda i: (i, 0)),
            pl.BlockSpec((_EXPERTS, _HID), lambda i: (0, 0)),
        ],
        out_specs=[
            pl.BlockSpec((2, _BR), lambda i: (0, i)),
            pl.BlockSpec((2, _BR), lambda i: (0, i)),
        ],
        out_shape=[
            jax.ShapeDtypeStruct((2, _ROWS), jnp.float32),
            jax.ShapeDtypeStruct((2, _ROWS), jnp.int32),
        ],
    )(hidden_states, weight)
    return (vals.T, idx.T)


# speedup vs baseline: 1.5891x; 1.0057x over previous
import functools

import jax
import jax.numpy as jnp
from jax.experimental import pallas as pl

_ROWS = 16384
_HID = 2048
_EXPERTS = 64
_BR = 1024


def _router_kernel(x_ref, w_ref, val_ref, idx_ref):
    lg = jax.lax.dot_general(
        w_ref[...], x_ref[...], (((1,), (1,)), ((), ())),
        preferred_element_type=jnp.float32,
    )
    iota = jax.lax.broadcasted_iota(jnp.int32, lg.shape, 0)
    m1 = jnp.max(lg, axis=0, keepdims=True)
    i1 = jnp.min(jnp.where(lg == m1, iota, _EXPERTS), axis=0, keepdims=True)
    masked = jnp.where(iota == i1, -jnp.inf, lg)
    m2 = jnp.max(masked, axis=0, keepdims=True)
    i2 = jnp.min(jnp.where(masked == m2, iota, _EXPERTS), axis=0, keepdims=True)
    e2 = jnp.exp(m2 - m1)
    inv = 1.0 / (1.0 + e2)
    val_ref[...] = jnp.concatenate([inv, e2 * inv], axis=0)
    idx_ref[...] = jnp.concatenate([i1, i2], axis=0)


@jax.jit
def kernel(hidden_states, weight):
    grid = (_ROWS // _BR,)
    vals, idx = pl.pallas_call(
        _router_kernel,
        grid=grid,
        in_specs=[
            pl.BlockSpec((_BR, _HID), lambda i: (i, 0)),
            pl.BlockSpec((_EXPERTS, _HID), lambda i: (0, 0)),
        ],
        out_specs=[
            pl.BlockSpec((2, _BR), lambda i: (0, i)),
            pl.BlockSpec((2, _BR), lambda i: (0, i)),
        ],
        out_shape=[
            jax.ShapeDtypeStruct((2, _ROWS), jnp.float32),
            jax.ShapeDtypeStruct((2, _ROWS), jnp.int32),
        ],
    )(hidden_states, weight)
    return (vals.T, idx.T)
